# Initial kernel scaffold; baseline (speedup 1.0000x reference)
import jax
import jax.numpy as jnp
from jax.experimental import pallas as pl


def kernel(x, edge_index, batch, W1, b1, W2, b2, Wlin, blin):
    raise NotImplementedError("write your pallas kernel here")


# speedup vs baseline: 17.2997x; 17.2997x over previous
import functools
import jax
import jax.numpy as jnp
from jax import lax
from jax.experimental import pallas as pl
from jax.experimental.pallas import tpu as pltpu
from jax.experimental.pallas import tpu_sc as plsc

N = 10000
E = 320000
D = 128
G = 64
NCLS = 2

EC = 128
NCHUNK = E // EC
NC = 2
NS = 16
NW = NC * NS
NPAD = 10240
ROWS_PER_TILE = NPAD // NS


def _fill(ref, val, n16):
    v = jnp.full((16,), val, ref.dtype)

    def body(i, _):
        ref[pl.ds(i * 16, 16)] = v
        return 0

    lax.fori_loop(0, n16, body, 0)


def _fill2d(ref, val, nrows):
    v = jnp.full((16,), val, ref.dtype)

    def body(i, _):
        r = i // 8
        c = (i % 8) * 16
        ref[r, pl.ds(c, 16)] = v
        return 0

    lax.fori_loop(0, nrows * 8, body, 0)


def _deg_body(dst2d, deg_out, deg_sh, idx_v, ones_v, zbuf):
    cid = lax.axis_index("c")
    sid = lax.axis_index("s")
    wid = sid * NC + cid

    _fill(ones_v, 1.0, EC // 16)
    _fill(zbuf, 0.0, 40)
    pltpu.sync_copy(zbuf, deg_sh.at[pl.ds(sid * 640, 640)])
    plsc.subcore_barrier()

    nche = (NCHUNK - wid + NW - 1) // NW

    def body(i, _):
        c = wid + i * NW
        pltpu.sync_copy(dst2d.at[c], idx_v)
        pltpu.sync_copy(ones_v, deg_sh.at[idx_v], add=True)
        return 0

    lax.fori_loop(0, nche, body, 0)
    plsc.subcore_barrier()

    pltpu.sync_copy(deg_sh.at[pl.ds(sid * 640, 640)],
                    deg_out.at[cid, pl.ds(sid * 640, 640)])


def _agg_body(y_hbm, src2d, dst2d, z_out, z_sh, src_v, dst_v, rows_v, sem):
    cid = lax.axis_index("c")
    sid = lax.axis_index("s")
    wid = sid * NC + cid

    _fill2d(rows_v, 0.0, EC)

    def zbody(k, _):
        pltpu.sync_copy(rows_v,
                        z_sh.at[pl.ds(sid * ROWS_PER_TILE + k * EC, EC)])
        return 0

    lax.fori_loop(0, ROWS_PER_TILE // EC, zbody, 0)
    plsc.subcore_barrier()

    nche = (NCHUNK - wid + NW - 1) // NW

    def body(i, _):
        c = wid + i * NW
        pltpu.sync_copy(src2d.at[c], src_v)
        pltpu.sync_copy(dst2d.at[c], dst_v)
        pltpu.async_copy(y_hbm.at[src_v], rows_v, sem).wait()
        pltpu.sync_copy(rows_v, z_sh.at[dst_v], add=True)
        return 0

    lax.fori_loop(0, nche, body, 0)
    plsc.subcore_barrier()

    pltpu.sync_copy(
        z_sh.at[pl.ds(sid * ROWS_PER_TILE, ROWS_PER_TILE)],
        z_out.at[cid, pl.ds(sid * ROWS_PER_TILE, ROWS_PER_TILE)])


def _tc1(x_ref, w1_ref, degp_ref, y1_ref, dinv_ref):
    deg = degp_ref[0, :N] + degp_ref[1, :N] + 1.0
    dinv = lax.rsqrt(deg)
    dinv_ref[...] = dinv
    xw = jnp.dot(x_ref[...], w1_ref[...], preferred_element_type=jnp.float32)
    y1_ref[...] = xw * dinv[:, None]


def _tc2(zp_ref, y1_ref, dinv_ref, b1_ref, w2_ref, y2_ref):
    dinv = dinv_ref[...]
    z = zp_ref[0, :N] + zp_ref[1, :N] + y1_ref[...]
    h = jnp.maximum(z * dinv[:, None] + b1_ref[...][None, :], 0.0)
    y2_ref[...] = jnp.dot(h, w2_ref[...],
                          preferred_element_type=jnp.float32) * dinv[:, None]


def _tc3(zp_ref, y2_ref, dinv_ref, b2_ref, batch_ref, wlin_ref, blin_ref,
         out_ref):
    dinv = dinv_ref[...]
    z = zp_ref[0, :N] + zp_ref[1, :N] + y2_ref[...]
    h = jnp.maximum(z * dinv[:, None] + b2_ref[...][None, :], 0.0)
    gids = lax.broadcasted_iota(jnp.int32, (G, N), 0)
    onehot = (gids == batch_ref[...][None, :]).astype(jnp.float32)
    sums = jnp.dot(onehot, h, preferred_element_type=jnp.float32)
    cnt = jnp.sum(onehot, axis=1, keepdims=True)
    pooled = sums / jnp.maximum(cnt, 1.0)
    out_ref[...] = (jnp.dot(pooled, wlin_ref[...],
                            preferred_element_type=jnp.float32)
                    + blin_ref[...][None, :])


@functools.cache
def _get_sc_kernels():
    mesh = plsc.VectorSubcoreMesh(core_axis_name="c", subcore_axis_name="s",
                                  num_cores=NC, num_subcores=NS)
    deg_kernel = pl.kernel(
        _deg_body,
        out_type=jax.ShapeDtypeStruct((NC, 10240), jnp.float32),
        mesh=mesh,
        scratch_types=[
            pltpu.VMEM_SHARED((10240,), jnp.float32),
            pltpu.VMEM((EC,), jnp.int32),
            pltpu.VMEM((EC,), jnp.float32),
            pltpu.VMEM((640,), jnp.float32),
        ],
    )
    agg_kernel = pl.kernel(
        _agg_body,
        out_type=jax.ShapeDtypeStruct((NC, NPAD, D), jnp.float32),
        mesh=mesh,
        scratch_types=[
            pltpu.VMEM_SHARED((NPAD, D), jnp.float32),
            pltpu.VMEM((EC,), jnp.int32),
            pltpu.VMEM((EC,), jnp.int32),
            pltpu.VMEM((EC, D), jnp.float32),
            pltpu.SemaphoreType.DMA,
        ],
    )
    return deg_kernel, agg_kernel


def kernel(x, edge_index, batch, W1, b1, W2, b2, Wlin, blin):
    _deg_kernel, _agg_kernel = _get_sc_kernels()
    src2d = edge_index[0].reshape(NCHUNK, EC)
    dst2d = edge_index[1].reshape(NCHUNK, EC)

    degp = _deg_kernel(dst2d)

    y1, dinv = pl.pallas_call(
        _tc1,
        out_shape=(jax.ShapeDtypeStruct((N, D), jnp.float32),
                   jax.ShapeDtypeStruct((N,), jnp.float32)),
    )(x, W1, degp)

    zp1 = _agg_kernel(y1, src2d, dst2d)

    y2 = pl.pallas_call(
        _tc2,
        out_shape=jax.ShapeDtypeStruct((N, D), jnp.float32),
    )(zp1, y1, dinv, b1, W2)

    zp2 = _agg_kernel(y2, src2d, dst2d)

    out = pl.pallas_call(
        _tc3,
        out_shape=jax.ShapeDtypeStruct((G, NCLS), jnp.float32),
    )(zp2, y2, dinv, b2, batch, Wlin, blin)

    return out
